# Initial kernel scaffold; baseline (speedup 1.0000x reference)
import jax
import jax.numpy as jnp
from jax.experimental import pallas as pl


def kernel(output, target):
    raise NotImplementedError("write your pallas kernel here")


# speedup vs baseline: 4.6527x; 4.6527x over previous
import functools

import jax
import jax.numpy as jnp
from jax.experimental import pallas as pl
from jax.experimental.pallas import tpu as pltpu


def _argmax_c(x):
    c = x.shape[0]
    val = x[0]
    idx = jnp.zeros(val.shape, dtype=jnp.int32)
    for k in range(1, c):
        gt = x[k] > val
        val = jnp.where(gt, x[k], val)
        idx = jnp.where(gt, jnp.int32(k), idx)
    return idx


def _iou_body(out_ref, tgt_ref, score_ref, acc_ref, *, nsteps, cls_num):
    step = pl.program_id(0) * pl.num_programs(1) + pl.program_id(1)

    @pl.when(step == 0)
    def _init():
        acc_ref[...] = jnp.zeros_like(acc_ref)

    o = out_ref[0]
    t = tgt_ref[0]
    oi = _argmax_c(o)
    ti = _argmax_c(t)
    matched = jnp.where(oi == ti, ti, jnp.int32(-1))

    rows = jax.lax.broadcasted_iota(jnp.int32, acc_ref.shape, 0)
    lanes = jax.lax.broadcasted_iota(jnp.int32, acc_ref.shape, 1)
    upd = jnp.zeros(acc_ref.shape, dtype=jnp.float32)
    for c in range(cls_num):
        tp_c = jnp.sum((matched == c).astype(jnp.float32))
        co_c = jnp.sum((oi == c).astype(jnp.float32))
        ct_c = jnp.sum((ti == c).astype(jnp.float32))
        at_c = lanes == c
        upd = upd + jnp.where((rows == 0) & at_c, tp_c, 0.0)
        upd = upd + jnp.where((rows == 1) & at_c, co_c, 0.0)
        upd = upd + jnp.where((rows == 2) & at_c, ct_c, 0.0)
    acc_ref[...] += upd

    @pl.when(step == nsteps - 1)
    def _finish():
        acc = acc_ref[...]
        tp = acc[0:1, :]
        denom = acc[1:2, :] + acc[2:3, :] - tp
        iou = jnp.where(denom > 0.0, tp / denom, 0.0)
        score_ref[...] = jnp.sum(iou, keepdims=True) / jnp.float32(cls_num)


def kernel(output, target):
    b, c, h, w = output.shape
    blk_r = 128
    n_r = h // blk_r
    nsteps = b * n_r

    body = functools.partial(_iou_body, nsteps=nsteps, cls_num=c)
    in_spec = pl.BlockSpec((1, c, blk_r, w), lambda i, r: (i, 0, r, 0))
    score = pl.pallas_call(
        body,
        grid=(b, n_r),
        in_specs=[in_spec, in_spec],
        out_specs=pl.BlockSpec((1, 1), lambda i, r: (0, 0)),
        out_shape=jax.ShapeDtypeStruct((1, 1), jnp.float32),
        scratch_shapes=[pltpu.VMEM((3, 32), jnp.float32)],
    )(output, target)
    return score[0, 0]


# speedup vs baseline: 5.0669x; 1.0890x over previous
import functools

import jax
import jax.numpy as jnp
from jax.experimental import pallas as pl
from jax.experimental.pallas import tpu as pltpu

_SUBROWS = 32


def _argmax_sub(ref, r0, sr):
    c = ref.shape[1]
    val = ref[0, 0, pl.ds(r0, sr), :]
    idx = jnp.zeros(val.shape, dtype=jnp.int32)
    for k in range(1, c):
        cur = ref[0, k, pl.ds(r0, sr), :]
        gt = cur > val
        val = jnp.maximum(cur, val)
        idx = jnp.where(gt, jnp.int32(k), idx)
    return idx


def _iou_body(out_ref, tgt_ref, score_ref, acc_ref, *, nsteps, cls_num):
    step = pl.program_id(0) * pl.num_programs(1) + pl.program_id(1)

    @pl.when(step == 0)
    def _init():
        acc_ref[...] = jnp.zeros_like(acc_ref)

    blk_r = out_ref.shape[2]
    tp = [jnp.int32(0)] * cls_num
    co = [jnp.int32(0)] * cls_num
    ct = [jnp.int32(0)] * cls_num
    for r0 in range(0, blk_r, _SUBROWS):
        oi = _argmax_sub(out_ref, r0, _SUBROWS)
        ti = _argmax_sub(tgt_ref, r0, _SUBROWS)
        for c in range(cls_num):
            mo = oi == c
            mt = ti == c
            tp[c] = tp[c] + jnp.sum(mo & mt)
            co[c] = co[c] + jnp.sum(mo)
            ct[c] = ct[c] + jnp.sum(mt)

    rows = jax.lax.broadcasted_iota(jnp.int32, acc_ref.shape, 0)
    lanes = jax.lax.broadcasted_iota(jnp.int32, acc_ref.shape, 1)
    upd = jnp.zeros(acc_ref.shape, dtype=jnp.float32)
    for c in range(cls_num):
        at_c = lanes == c
        upd = upd + jnp.where((rows == 0) & at_c, tp[c].astype(jnp.float32), 0.0)
        upd = upd + jnp.where((rows == 1) & at_c, co[c].astype(jnp.float32), 0.0)
        upd = upd + jnp.where((rows == 2) & at_c, ct[c].astype(jnp.float32), 0.0)
    acc_ref[...] += upd

    @pl.when(step == nsteps - 1)
    def _finish():
        acc = acc_ref[...]
        tps = acc[0:1, :]
        denom = acc[1:2, :] + acc[2:3, :] - tps
        iou = jnp.where(denom > 0.0, tps / denom, 0.0)
        score_ref[...] = jnp.sum(iou, keepdims=True) / jnp.float32(cls_num)


def kernel(output, target):
    b, c, h, w = output.shape
    blk_r = 128
    n_r = h // blk_r
    nsteps = b * n_r

    body = functools.partial(_iou_body, nsteps=nsteps, cls_num=c)
    in_spec = pl.BlockSpec((1, c, blk_r, w), lambda i, r: (i, 0, r, 0))
    score = pl.pallas_call(
        body,
        grid=(b, n_r),
        in_specs=[in_spec, in_spec],
        out_specs=pl.BlockSpec((1, 1), lambda i, r: (0, 0)),
        out_shape=jax.ShapeDtypeStruct((1, 1), jnp.float32),
        scratch_shapes=[pltpu.VMEM((3, 32), jnp.float32)],
    )(output, target)
    return score[0, 0]


# speedup vs baseline: 5.1981x; 1.0259x over previous
import functools

import jax
import jax.numpy as jnp
from jax.experimental import pallas as pl
from jax.experimental.pallas import tpu as pltpu

_SUBROWS = 32


def _argmax_sub(ref, r0, sr):
    c = ref.shape[1]
    val = ref[0, 0, pl.ds(r0, sr), :]
    idx = jnp.zeros(val.shape, dtype=jnp.int32)
    for k in range(1, c):
        cur = ref[0, k, pl.ds(r0, sr), :]
        gt = cur > val
        val = jnp.maximum(cur, val)
        idx = jnp.where(gt, jnp.int32(k), idx)
    return idx


def _iou_body(out_ref, tgt_ref, score_ref, acc_ref, *, nsteps, cls_num):
    step = pl.program_id(0) * pl.num_programs(1) + pl.program_id(1)

    @pl.when(step == 0)
    def _init():
        acc_ref[...] = jnp.zeros_like(acc_ref)

    blk_r = out_ref.shape[2]
    tp = [jnp.int32(0)] * cls_num
    co = [jnp.int32(0)] * cls_num
    ct = [jnp.int32(0)] * cls_num
    for r0 in range(0, blk_r, _SUBROWS):
        oi = _argmax_sub(out_ref, r0, _SUBROWS)
        ti = _argmax_sub(tgt_ref, r0, _SUBROWS)
        for c in range(cls_num):
            mo = oi == c
            mt = ti == c
            tp[c] = tp[c] + jnp.sum(mo & mt)
            co[c] = co[c] + jnp.sum(mo)
            ct[c] = ct[c] + jnp.sum(mt)

    rows = jax.lax.broadcasted_iota(jnp.int32, acc_ref.shape, 0)
    lanes = jax.lax.broadcasted_iota(jnp.int32, acc_ref.shape, 1)
    upd = jnp.zeros(acc_ref.shape, dtype=jnp.float32)
    for c in range(cls_num):
        at_c = lanes == c
        upd = upd + jnp.where((rows == 0) & at_c, tp[c].astype(jnp.float32), 0.0)
        upd = upd + jnp.where((rows == 1) & at_c, co[c].astype(jnp.float32), 0.0)
        upd = upd + jnp.where((rows == 2) & at_c, ct[c].astype(jnp.float32), 0.0)
    acc_ref[...] += upd

    @pl.when(step == nsteps - 1)
    def _finish():
        acc = acc_ref[...]
        tps = acc[0:1, :]
        denom = acc[1:2, :] + acc[2:3, :] - tps
        iou = jnp.where(denom > 0.0, tps / denom, 0.0)
        score_ref[...] = jnp.sum(iou, keepdims=True) / jnp.float32(cls_num)


def kernel(output, target):
    b, c, h, w = output.shape
    blk_r = 256
    n_r = h // blk_r
    nsteps = b * n_r

    body = functools.partial(_iou_body, nsteps=nsteps, cls_num=c)
    in_spec = pl.BlockSpec((1, c, blk_r, w), lambda i, r: (i, 0, r, 0))
    score = pl.pallas_call(
        body,
        grid=(b, n_r),
        in_specs=[in_spec, in_spec],
        out_specs=pl.BlockSpec((1, 1), lambda i, r: (0, 0)),
        out_shape=jax.ShapeDtypeStruct((1, 1), jnp.float32),
        scratch_shapes=[pltpu.VMEM((3, 32), jnp.float32)],
    )(output, target)
    return score[0, 0]
